# baseline (device time: 14471 ns/iter reference)
import jax
import jax.numpy as jnp
from jax import lax
from jax.experimental import pallas as pl
from jax.experimental.pallas import tpu as pltpu

N_DEV = 8
N_ROWS_GLOBAL = 32768
CHUNK = 512


def kernel(x):
    m_per, n = x.shape
    n_chunks = m_per // CHUNK

    def body(x_hbm, out_hbm, xbuf, comm_ref, out_vmem,
             copy_sems, send_sems, recv_sems, credit_sems, out_sem):
        me = lax.axis_index("i")

        barrier_sem = pltpu.get_barrier_semaphore()
        pl.semaphore_signal(barrier_sem, inc=1)
        pl.semaphore_wait(barrier_sem, 1)

        for k in range(1, N_DEV):
            pl.semaphore_signal(
                credit_sems.at[N_DEV - k],
                inc=1,
                device_id=(lax.rem(me + k, N_DEV),),
                device_id_type=pl.DeviceIdType.MESH,
            )

        def chunk_copy(c, slot):
            return pltpu.make_async_copy(
                x_hbm.at[pl.ds(c * CHUNK, CHUNK), :],
                xbuf.at[slot],
                copy_sems.at[slot],
            )

        chunk_copy(0, 0).start()
        acc = None
        for c in range(n_chunks):
            slot = c % 2
            if c + 1 < n_chunks:
                chunk_copy(c + 1, (c + 1) % 2).start()
            chunk_copy(c, slot).wait()
            part = jnp.sum(xbuf[slot], axis=0)
            acc = part if acc is None else acc + part
        comm_ref[0, :] = acc

        sends = []
        for k in range(1, N_DEV):
            pl.semaphore_wait(credit_sems.at[k], 1)
            rdma = pltpu.make_async_remote_copy(
                src_ref=comm_ref.at[0],
                dst_ref=comm_ref.at[N_DEV - k],
                send_sem=send_sems.at[k - 1],
                recv_sem=recv_sems.at[N_DEV - k],
                device_id=(lax.rem(me + k, N_DEV),),
                device_id_type=pl.DeviceIdType.MESH,
            )
            rdma.start()
            sends.append(rdma)

        for j in range(1, N_DEV):
            recv = pltpu.make_async_remote_copy(
                src_ref=comm_ref.at[0],
                dst_ref=comm_ref.at[j],
                send_sem=send_sems.at[0],
                recv_sem=recv_sems.at[j],
                device_id=(lax.rem(me + j, N_DEV),),
                device_id_type=pl.DeviceIdType.MESH,
            )
            recv.wait_recv()

        for rdma in sends:
            rdma.wait_send()

        total = jnp.sum(comm_ref[:, :], axis=0)
        out_vmem[0, :] = total * (1.0 / N_ROWS_GLOBAL)
        out_copy = pltpu.make_async_copy(out_vmem, out_hbm, out_sem)
        out_copy.start()
        out_copy.wait()

    return pl.pallas_call(
        body,
        out_shape=jax.ShapeDtypeStruct((1, n), jnp.float32),
        in_specs=[pl.BlockSpec(memory_space=pltpu.MemorySpace.HBM)],
        out_specs=pl.BlockSpec(memory_space=pltpu.MemorySpace.HBM),
        scratch_shapes=[
            pltpu.VMEM((2, CHUNK, n), jnp.float32),
            pltpu.VMEM((N_DEV, n), jnp.float32),
            pltpu.VMEM((1, n), jnp.float32),
            pltpu.SemaphoreType.DMA((2,)),
            pltpu.SemaphoreType.DMA((N_DEV - 1,)),
            pltpu.SemaphoreType.DMA((N_DEV,)),
            pltpu.SemaphoreType.REGULAR((N_DEV,)),
            pltpu.SemaphoreType.DMA,
        ],
        compiler_params=pltpu.CompilerParams(collective_id=0),
    )(x)
